# baseline (speedup 1.0000x reference)
import functools
import math

import jax
import jax.numpy as jnp
from jax import lax
from jax.experimental import pallas as pl
from jax.experimental.pallas import tpu as pltpu
from jax.experimental.pallas import tpu_sc as plsc

VOCAB = 100000
D = 1024
SEQ = 4096
BATCH = 4
B = BATCH * SEQ
SCALE = math.sqrt(D)

NC = 2
NS = 16
NW = NC * NS
NSEQ_W = SEQ // NW
ROWS_PER_W = B // NW
CK = 32
NCHUNK = ROWS_PER_W // CK
NPAIR = NCHUNK // 2
L = 16
SL = 8
LN = D // SL
VPL = LN // L

_mesh = plsc.VectorSubcoreMesh(
    core_axis_name="c", subcore_axis_name="s", num_cores=NC, num_subcores=NS
)


@functools.partial(
    pl.kernel,
    out_type=jax.ShapeDtypeStruct((B, SL, LN), jnp.float32),
    mesh=_mesh,
    scratch_types=[
        pltpu.VMEM((ROWS_PER_W,), jnp.int32),
        pltpu.VMEM((2 * CK, SL, LN), jnp.float32),
        pltpu.VMEM((CK, SL, LN), jnp.float32),
        pltpu.SemaphoreType.DMA,
        pltpu.SemaphoreType.DMA,
        pltpu.SemaphoreType.DMA,
    ],
)
def _embed_sc(x_hbm, tok_hbm, pos_hbm, out_hbm, idx_v, rows_v, pos_v,
              gsem0, gsem1, ssem):
    wid = lax.axis_index("s") * NC + lax.axis_index("c")
    s0 = wid * NSEQ_W

    for b in range(BATCH):
        pltpu.sync_copy(
            x_hbm.at[pl.ds(b * SEQ + s0, NSEQ_W)],
            idx_v.at[pl.ds(b * NSEQ_W, NSEQ_W)],
        )

    def idx_off(c):
        return (c % BATCH) * NSEQ_W + (c // BATCH) * CK

    def out_off(c):
        return (c % BATCH) * SEQ + s0 + (c // BATCH) * CK

    def issue_gather(c, boff, sem):
        pltpu.async_copy(
            tok_hbm.at[idx_v.at[pl.ds(idx_off(c), CK)]],
            rows_v.at[pl.ds(boff, CK)],
            sem,
        )

    def wait_gather(boff, sem):
        pltpu.make_async_copy(
            tok_hbm.at[pl.ds(0, CK)], rows_v.at[pl.ds(boff, CK)], sem
        ).wait()

    def drain_store():
        pltpu.make_async_copy(
            rows_v.at[pl.ds(0, CK)], out_hbm.at[pl.ds(0, CK)], ssem
        ).wait()

    def compute(boff):
        @pl.loop(0, CK)
        def _row(r):
            for u in range(SL):
                for j in range(VPL):
                    sl = pl.ds(j * L, L)
                    rows_v[boff + r, u, sl] = (
                        rows_v[boff + r, u, sl] * SCALE + pos_v[r, u, sl]
                    )

    def store(c, boff):
        pltpu.async_copy(
            rows_v.at[pl.ds(boff, CK)], out_hbm.at[pl.ds(out_off(c), CK)], ssem
        )

    issue_gather(0, 0, gsem0)

    @pl.loop(0, NPAIR)
    def _pair(t):
        c0 = 2 * t
        c1 = c0 + 1

        @pl.when(t >= 1)
        def _():
            drain_store()

        issue_gather(c1, CK, gsem1)

        @pl.when(t % 2 == 0)
        def _():
            pltpu.sync_copy(
                pos_hbm.at[pl.ds(s0 + (t // 2) * CK, CK)], pos_v
            )

        wait_gather(0, gsem0)
        compute(0)
        store(c0, 0)

        @pl.when(t < NPAIR - 1)
        def _():
            drain_store()
            issue_gather(c0 + 2, 0, gsem0)

        wait_gather(CK, gsem1)
        compute(CK)
        store(c1, CK)

    drain_store()
    drain_store()


def kernel(x, token_table, pos_table):
    out = _embed_sc(
        x.reshape(-1),
        token_table.reshape(VOCAB, SL, LN),
        pos_table.reshape(SEQ, SL, LN),
    )
    return out.reshape(BATCH, SEQ, D)


# speedup vs baseline: 3.1645x; 3.1645x over previous
import functools
import math

import jax
import jax.numpy as jnp
from jax import lax
from jax.experimental import pallas as pl
from jax.experimental.pallas import tpu as pltpu
from jax.experimental.pallas import tpu_sc as plsc

VOCAB = 100000
D = 1024
SEQ = 4096
BATCH = 4
B = BATCH * SEQ
SCALE = math.sqrt(D)

NC = 2
NS = 16
NW = NC * NS
NSEQ_W = SEQ // NW
ROWS_PER_W = B // NW
HK = 16
CK = 2 * HK
NCHUNK = ROWS_PER_W // CK
NBUF = 3
L = 16
VPR = D // L

_mesh = plsc.VectorSubcoreMesh(
    core_axis_name="c", subcore_axis_name="s", num_cores=NC, num_subcores=NS
)


@functools.partial(
    pl.kernel,
    out_type=jax.ShapeDtypeStruct((B, D), jnp.float32),
    mesh=_mesh,
    scratch_types=[
        pltpu.VMEM((ROWS_PER_W,), jnp.int32),
        pltpu.VMEM((NBUF * CK, D), jnp.float32),
        pltpu.VMEM((HK, D), jnp.float32),
        pltpu.SemaphoreType.DMA,
        pltpu.SemaphoreType.DMA,
        pltpu.SemaphoreType.DMA,
        pltpu.SemaphoreType.DMA,
        pltpu.SemaphoreType.DMA,
    ],
)
def _embed_sc(x_hbm, tok_hbm, pos_hbm, out_hbm, idx_v, rows_v, pos_v,
              g0, g1, g2, psem, ssem):
    gsems = (g0, g1, g2)
    wid = lax.axis_index("s") * NC + lax.axis_index("c")
    s0 = wid * NSEQ_W

    for b in range(BATCH):
        pltpu.sync_copy(
            x_hbm.at[pl.ds(b * SEQ + s0, NSEQ_W)],
            idx_v.at[pl.ds(b * NSEQ_W, NSEQ_W)],
        )

    def issue_gather(c, buf):
        base = (c % 2) * (2 * NSEQ_W) + (c // 2) * HK
        for h in range(2):
            pltpu.async_copy(
                tok_hbm.at[idx_v.at[pl.ds(base + h * NSEQ_W, HK)]],
                rows_v.at[pl.ds(buf * CK + h * HK, HK)],
                gsems[buf],
            )

    def wait_gather(buf):
        pltpu.make_async_copy(
            tok_hbm.at[pl.ds(0, CK)],
            rows_v.at[pl.ds(buf * CK, CK)],
            gsems[buf],
        ).wait()

    def issue_pos(k):
        pltpu.async_copy(pos_hbm.at[pl.ds(s0 + k * HK, HK)], pos_v, psem)

    def wait_pos():
        pltpu.make_async_copy(pos_hbm.at[pl.ds(0, HK)], pos_v, psem).wait()

    def store(c, buf):
        ro = (c % 2) * (2 * SEQ) + s0 + (c // 2) * HK
        for h in range(2):
            pltpu.async_copy(
                rows_v.at[pl.ds(buf * CK + h * HK, HK)],
                out_hbm.at[pl.ds(ro + h * SEQ, HK)],
                ssem,
            )

    def drain_store():
        pltpu.make_async_copy(
            rows_v.at[pl.ds(0, CK)], out_hbm.at[pl.ds(0, CK)], ssem
        ).wait()

    def compute(buf):
        for half in range(2):
            hoff = buf * CK + half * HK

            @pl.loop(0, HK)
            def _row(r):
                for j in range(VPR):
                    sl = pl.ds(j * L, L)
                    rows_v[hoff + r, sl] = (
                        rows_v[hoff + r, sl] * SCALE + pos_v[r, sl]
                    )

    def chunk_iter(c, buf, has_next, guard_drain):
        wait_gather(buf)
        if has_next:
            issue_gather(c + 1, (buf + 1) % NBUF)

        @pl.when(c % 2 == 0)
        def _():
            wait_pos()

        compute(buf)
        store(c, buf)

        @pl.when(c % 2 == 1)
        def _():
            issue_pos(c // 2 + 1)

        if guard_drain is None:
            drain_store()
        else:
            @pl.when(guard_drain)
            def _():
                drain_store()

    issue_pos(0)
    issue_gather(0, 0)

    @pl.loop(0, NCHUNK // NBUF)
    def _t(t):
        c0 = NBUF * t
        chunk_iter(c0, 0, True, t >= 1)
        chunk_iter(c0 + 1, 1, True, None)
        chunk_iter(c0 + 2, 2, True, None)

    wait_gather(0)
    compute(0)
    store(NCHUNK - 1, 0)
    drain_store()
    drain_store()


def kernel(x, token_table, pos_table):
    out = _embed_sc(x.reshape(-1), token_table, pos_table)
    return out.reshape(BATCH, SEQ, D)


# speedup vs baseline: 5.9787x; 1.8893x over previous
import functools
import math

import jax
import jax.numpy as jnp
from jax import lax
from jax.experimental import pallas as pl
from jax.experimental.pallas import tpu as pltpu
from jax.experimental.pallas import tpu_sc as plsc

VOCAB = 100000
D = 1024
SEQ = 4096
BATCH = 4
B = BATCH * SEQ
SCALE = math.sqrt(D)

NC = 2
NS = 16
NW = NC * NS
NSEQ_W = SEQ // NW
ROWS_PER_W = B // NW
PK = 8
CK = BATCH * PK
NCHUNK = ROWS_PER_W // CK
NPAIR = NCHUNK // 2
L = 16
VPR = D // L

_mesh = plsc.VectorSubcoreMesh(
    core_axis_name="c", subcore_axis_name="s", num_cores=NC, num_subcores=NS
)


@functools.partial(
    pl.kernel,
    out_type=jax.ShapeDtypeStruct((B, D), jnp.float32),
    mesh=_mesh,
    scratch_types=[
        pltpu.VMEM((ROWS_PER_W,), jnp.int32),
        pltpu.VMEM((2 * CK, D), jnp.float32),
        pltpu.VMEM((2 * PK, D), jnp.float32),
        pltpu.SemaphoreType.DMA,
        pltpu.SemaphoreType.DMA,
        pltpu.SemaphoreType.DMA,
        pltpu.SemaphoreType.DMA,
        pltpu.SemaphoreType.DMA,
    ],
)
def _embed_sc(x_hbm, tok_hbm, pos_hbm, out_hbm, idx_v, rows_v, pos_v,
              gsem0, gsem1, psem0, psem1, ssem):
    wid = lax.axis_index("s") * NC + lax.axis_index("c")
    s0 = wid * NSEQ_W

    for b in range(BATCH):
        pltpu.sync_copy(
            x_hbm.at[pl.ds(b * SEQ + s0, NSEQ_W)],
            idx_v.at[pl.ds(b * NSEQ_W, NSEQ_W)],
        )

    def issue_gather(c, boff, sem):
        for b in range(BATCH):
            pltpu.async_copy(
                tok_hbm.at[idx_v.at[pl.ds(b * NSEQ_W + c * PK, PK)]],
                rows_v.at[pl.ds(boff + b * PK, PK)],
                sem,
            )

    def wait_gather(boff, sem):
        pltpu.make_async_copy(
            tok_hbm.at[pl.ds(0, CK)], rows_v.at[pl.ds(boff, CK)], sem
        ).wait()

    def issue_pos(c, poff, sem):
        pltpu.async_copy(
            pos_hbm.at[pl.ds(s0 + c * PK, PK)],
            pos_v.at[pl.ds(poff, PK)],
            sem,
        )

    def wait_pos(poff, sem):
        pltpu.make_async_copy(
            pos_hbm.at[pl.ds(0, PK)], pos_v.at[pl.ds(poff, PK)], sem
        ).wait()

    def store(c, boff):
        for b in range(BATCH):
            pltpu.async_copy(
                rows_v.at[pl.ds(boff + b * PK, PK)],
                out_hbm.at[pl.ds(b * SEQ + s0 + c * PK, PK)],
                ssem,
            )

    def drain_store():
        pltpu.make_async_copy(
            rows_v.at[pl.ds(0, CK)], out_hbm.at[pl.ds(0, CK)], ssem
        ).wait()

    def compute(boff, poff):
        @pl.loop(0, PK)
        def _row(r):
            for j in range(VPR):
                sl = pl.ds(j * L, L)
                p = pos_v[poff + r, sl]
                for b in range(BATCH):
                    row = boff + b * PK + r
                    rows_v[row, sl] = rows_v[row, sl] * SCALE + p

    issue_pos(0, 0, psem0)
    issue_pos(1, PK, psem1)
    issue_gather(0, 0, gsem0)

    @pl.loop(0, NPAIR)
    def _pair(t):
        c0 = 2 * t
        c1 = c0 + 1

        @pl.when(t >= 1)
        def _():
            drain_store()

        issue_gather(c1, CK, gsem1)

        wait_pos(0, psem0)
        wait_gather(0, gsem0)
        compute(0, 0)
        store(c0, 0)

        @pl.when(t < NPAIR - 1)
        def _():
            issue_pos(c0 + 2, 0, psem0)
            drain_store()
            issue_gather(c0 + 2, 0, gsem0)

        wait_pos(PK, psem1)
        wait_gather(CK, gsem1)
        compute(CK, PK)
        store(c1, CK)

        @pl.when(t < NPAIR - 1)
        def _():
            issue_pos(c1 + 2, PK, psem1)

    drain_store()
    drain_store()


def kernel(x, token_table, pos_table):
    out = _embed_sc(x.reshape(-1), token_table, pos_table)
    return out.reshape(BATCH, SEQ, D)


# speedup vs baseline: 6.0768x; 1.0164x over previous
import functools
import math

import jax
import jax.numpy as jnp
from jax import lax
from jax.experimental import pallas as pl
from jax.experimental.pallas import tpu as pltpu
from jax.experimental.pallas import tpu_sc as plsc

VOCAB = 100000
D = 1024
SEQ = 4096
BATCH = 4
B = BATCH * SEQ
SCALE = math.sqrt(D)

NC = 2
NS = 16
NW = NC * NS
NSEQ_W = SEQ // NW
ROWS_PER_W = B // NW
PK = 8
CK = BATCH * PK
NCHUNK = ROWS_PER_W // CK
NPAIR = NCHUNK // 2
L = 16
VPR = D // L

_mesh = plsc.VectorSubcoreMesh(
    core_axis_name="c", subcore_axis_name="s", num_cores=NC, num_subcores=NS
)


@functools.partial(
    pl.kernel,
    out_type=jax.ShapeDtypeStruct((B, D), jnp.float32),
    mesh=_mesh,
    scratch_types=[
        pltpu.VMEM((ROWS_PER_W,), jnp.int32),
        pltpu.VMEM((2 * CK, D), jnp.float32),
        pltpu.VMEM((2 * PK, D), jnp.float32),
        pltpu.SemaphoreType.DMA,
        pltpu.SemaphoreType.DMA,
        pltpu.SemaphoreType.DMA,
        pltpu.SemaphoreType.DMA,
        pltpu.SemaphoreType.DMA,
    ],
)
def _embed_sc(x_hbm, tok_hbm, pos_hbm, out_hbm, idx_v, rows_v, pos_v,
              gsem0, gsem1, psem0, psem1, ssem):
    wid = lax.axis_index("s") * NC + lax.axis_index("c")
    s0 = wid * NSEQ_W

    pltpu.sync_copy(
        x_hbm.at[pl.ds(wid * ROWS_PER_W, ROWS_PER_W)], idx_v
    )

    def issue_gather(c, boff, sem):
        for b in range(BATCH):
            pltpu.async_copy(
                tok_hbm.at[idx_v.at[pl.ds(b * NSEQ_W + c * PK, PK)]],
                rows_v.at[pl.ds(boff + b * PK, PK)],
                sem,
            )

    def wait_gather(boff, sem):
        pltpu.make_async_copy(
            tok_hbm.at[pl.ds(0, CK)], rows_v.at[pl.ds(boff, CK)], sem
        ).wait()

    def issue_pos(c, poff, sem):
        pltpu.async_copy(
            pos_hbm.at[pl.ds(s0 + c * PK, PK)],
            pos_v.at[pl.ds(poff, PK)],
            sem,
        )

    def wait_pos(poff, sem):
        pltpu.make_async_copy(
            pos_hbm.at[pl.ds(0, PK)], pos_v.at[pl.ds(poff, PK)], sem
        ).wait()

    def store(c, boff):
        for b in range(BATCH):
            pltpu.async_copy(
                rows_v.at[pl.ds(boff + b * PK, PK)],
                out_hbm.at[pl.ds(b * SEQ + s0 + c * PK, PK)],
                ssem,
            )

    def drain_store():
        pltpu.make_async_copy(
            rows_v.at[pl.ds(0, CK)], out_hbm.at[pl.ds(0, CK)], ssem
        ).wait()

    def compute(boff, poff):
        @pl.loop(0, PK)
        def _row(r):
            for j in range(VPR):
                sl = pl.ds(j * L, L)
                p = pos_v[poff + r, sl]
                for b in range(BATCH):
                    row = boff + b * PK + r
                    rows_v[row, sl] = rows_v[row, sl] * SCALE + p

    issue_pos(0, 0, psem0)
    issue_pos(1, PK, psem1)
    issue_gather(0, 0, gsem0)

    @pl.loop(0, NPAIR)
    def _pair(t):
        c0 = 2 * t
        c1 = c0 + 1

        @pl.when(t >= 1)
        def _():
            drain_store()

        issue_gather(c1, CK, gsem1)

        wait_pos(0, psem0)
        wait_gather(0, gsem0)
        compute(0, 0)
        store(c0, 0)

        @pl.when(t < NPAIR - 1)
        def _():
            issue_pos(c0 + 2, 0, psem0)
            drain_store()
            issue_gather(c0 + 2, 0, gsem0)

        wait_pos(PK, psem1)
        wait_gather(CK, gsem1)
        compute(CK, PK)
        store(c1, CK)

        @pl.when(t < NPAIR - 1)
        def _():
            issue_pos(c1 + 2, PK, psem1)

    drain_store()
    drain_store()


def kernel(x, token_table, pos_table):
    x_arr = jnp.transpose(x.reshape(BATCH, NW, NSEQ_W), (1, 0, 2)).reshape(-1)
    out = _embed_sc(x_arr, token_table, pos_table)
    return out.reshape(BATCH, SEQ, D)
